# baseline (speedup 1.0000x reference)
import functools

import jax
import jax.numpy as jnp
from jax import lax
from jax.experimental import pallas as pl
from jax.experimental.pallas import tpu as pltpu
from jax.experimental.pallas import tpu_sc as plsc

NUM_FEATURES = 2
B = 2
S = 128
D = 64
REPS = 4
H = NUM_FEATURES * REPS
NW = 32
ROWS_PER_W = S // NW
NSLOT = 6
PREFETCH = 3
UNITS = [(f, b, r)
         for f in range(NUM_FEATURES)
         for b in range(B)
         for r in range(ROWS_PER_W)]


def _sc_kernel_body(idx_hbm, kt0, vt0, kt1, vt1, k_out, v_out, idx_v,
                    kb0, kb1, kb2, kb3, kb4, kb5,
                    vb0, vb1, vb2, vb3, vb4, vb5,
                    gs0, gs1, gs2, gs3, gs4, gs5,
                    ss0, ss1, ss2, ss3, ss4, ss5):
    wid = lax.axis_index("s") * 2 + lax.axis_index("c")
    kbufs = (kb0, kb1, kb2, kb3, kb4, kb5)
    vbufs = (vb0, vb1, vb2, vb3, vb4, vb5)
    gsems = (gs0, gs1, gs2, gs3, gs4, gs5)
    ssems = (ss0, ss1, ss2, ss3, ss4, ss5)
    k_tables = (kt0, kt1)
    v_tables = (vt0, vt1)
    pltpu.sync_copy(idx_hbm.at[wid], idx_v)
    gathers = [None] * NSLOT
    scatters = [None] * NSLOT

    def issue_gathers(t):
        f, b, r = UNITS[t]
        slot = t % NSLOT
        row = (f * B + b) * ROWS_PER_W + r
        gathers[slot] = [
            pltpu.async_copy(k_tables[f].at[idx_v.at[row]],
                             kbufs[slot].at[0], gsems[slot]),
            pltpu.async_copy(v_tables[f].at[idx_v.at[row]],
                             vbufs[slot].at[0], gsems[slot]),
        ]

    for t in range(PREFETCH):
        issue_gathers(t)
    for t in range(len(UNITS)):
        f, b, r = UNITS[t]
        slot = t % NSLOT
        for g in gathers[slot]:
            g.wait()
        row_out = wid * ROWS_PER_W + r
        ss = []
        for rep in range(REPS):
            h = REPS * f + rep
            ss.append(pltpu.async_copy(
                kbufs[slot], k_out.at[b, h, pl.ds(row_out, 1), :, :],
                ssems[slot]))
            ss.append(pltpu.async_copy(
                vbufs[slot], v_out.at[b, h, pl.ds(row_out, 1), :, :],
                ssems[slot]))
        scatters[slot] = ss
        nxt = t + PREFETCH
        if nxt < len(UNITS):
            nslot = nxt % NSLOT
            if scatters[nslot] is not None:
                for s in scatters[nslot]:
                    s.wait()
                scatters[nslot] = None
            issue_gathers(nxt)
    for ss in scatters:
        if ss is not None:
            for s in ss:
                s.wait()


@jax.jit
def _tree_rel_pos(idx_perm, kt0, vt0, kt1, vt1):
    out_sds = jax.ShapeDtypeStruct((B, H, S, S, D), jnp.float32)
    mesh = plsc.VectorSubcoreMesh(core_axis_name="c", subcore_axis_name="s")
    buf = pltpu.VMEM((1, S, D), jnp.float32)
    run = functools.partial(
        pl.kernel,
        out_type=[out_sds, out_sds],
        mesh=mesh,
        scratch_types=[pltpu.VMEM((NUM_FEATURES * B * ROWS_PER_W, S),
                                  jnp.int32)]
        + [buf] * (2 * NSLOT) + [pltpu.SemaphoreType.DMA] * (2 * NSLOT),
        compiler_params=pltpu.CompilerParams(use_tc_tiling_on_sc=False),
    )(_sc_kernel_body)
    return run(idx_perm, kt0, vt0, kt1, vt1)


def kernel(inputs, emb0, emb1):
    scale = float(D) ** 0.5
    idx_perm = jnp.transpose(
        inputs.reshape(NUM_FEATURES, B, NW, ROWS_PER_W, S),
        (2, 0, 1, 3, 4)).reshape(NW, NUM_FEATURES * B * ROWS_PER_W, S)
    kt0 = emb0[:, :D] * scale
    vt0 = emb0[:, D:] * scale
    kt1 = emb1[:, :D] * scale
    vt1 = emb1[:, D:] * scale
    k_out, v_out = _tree_rel_pos(idx_perm, kt0, vt0, kt1, vt1)
    return (k_out, v_out)


# speedup vs baseline: 1.1695x; 1.1695x over previous
import functools

import jax
import jax.numpy as jnp
from jax import lax
from jax.experimental import pallas as pl
from jax.experimental.pallas import tpu as pltpu
from jax.experimental.pallas import tpu_sc as plsc

NUM_FEATURES = 2
B = 2
S = 128
D = 64
VPAD = 64
REPS = 4
H = NUM_FEATURES * REPS
NW = 32
ROWS_PER_W = S // NW
NSLOT = 3
PREFETCH = 2
UNITS = [(f, b) for f in range(NUM_FEATURES) for b in range(B)]


def _sc_k_body(idx_hbm, kt0, kt1, k_out, idx_v, b0, b1, b2,
               gs0, gs1, gs2, ss0, ss1, ss2):
    wid = lax.axis_index("s") * 2 + lax.axis_index("c")
    bufs = (b0, b1, b2)
    gsems = (gs0, gs1, gs2)
    ssems = (ss0, ss1, ss2)
    tables = (kt0, kt1)
    pltpu.sync_copy(idx_hbm.at[wid], idx_v)
    gathers = [None] * NSLOT
    scatters = [None] * NSLOT

    def issue_gathers(t):
        f, b = UNITS[t]
        slot = t % NSLOT
        gathers[slot] = [
            pltpu.async_copy(tables[f].at[idx_v.at[(f * B + b) * ROWS_PER_W + c]],
                             bufs[slot].at[c], gsems[slot])
            for c in range(ROWS_PER_W)
        ]

    for t in range(PREFETCH):
        issue_gathers(t)
    i0 = wid * ROWS_PER_W
    for t in range(len(UNITS)):
        f, b = UNITS[t]
        slot = t % NSLOT
        for g in gathers[slot]:
            g.wait()
        scatters[slot] = [
            pltpu.async_copy(bufs[slot],
                             k_out.at[b, REPS * f + rep, pl.ds(i0, ROWS_PER_W), :, :],
                             ssems[slot])
            for rep in range(REPS)
        ]
        nxt = t + PREFETCH
        if nxt < len(UNITS):
            nslot = nxt % NSLOT
            if scatters[nslot] is not None:
                for s in scatters[nslot]:
                    s.wait()
                scatters[nslot] = None
            issue_gathers(nxt)
    for ss in scatters:
        if ss is not None:
            for s in ss:
                s.wait()


def _tc_v_body(idx_ref, vtab_ref, v_ref):
    iota_v = lax.broadcasted_iota(jnp.int32, (VPAD, 1), 0)
    for f in range(NUM_FEATURES):
        vtab = vtab_ref[f]
        for s in range(8):
            row = idx_ref[f, 0, s, :].reshape(1, S)
            oh = (row == iota_v).astype(jnp.float32)
            r = lax.dot_general(oh, vtab, (((0,), (0,)), ((), ())),
                                preferred_element_type=jnp.float32)
            for rep in range(REPS):
                v_ref[0, REPS * f + rep, s] = r


@jax.jit
def _tree_rel_pos(idx_perm, idx_raw, kt0, kt1, vtabs):
    out_sds = jax.ShapeDtypeStruct((B, H, S, S, D), jnp.float32)
    mesh = plsc.VectorSubcoreMesh(core_axis_name="c", subcore_axis_name="s")
    buf = pltpu.VMEM((ROWS_PER_W, S, D), jnp.float32)
    sc_run = functools.partial(
        pl.kernel,
        out_type=out_sds,
        mesh=mesh,
        scratch_types=[pltpu.VMEM((NUM_FEATURES * B * ROWS_PER_W, S),
                                  jnp.int32)]
        + [buf] * NSLOT + [pltpu.SemaphoreType.DMA] * (2 * NSLOT),
        compiler_params=pltpu.CompilerParams(use_tc_tiling_on_sc=False),
    )(_sc_k_body)
    k_out = sc_run(idx_perm, kt0, kt1)

    v_out = pl.pallas_call(
        _tc_v_body,
        grid=(B, S // 8),
        in_specs=[
            pl.BlockSpec((NUM_FEATURES, 1, 8, S), lambda b, i: (0, b, i, 0)),
            pl.BlockSpec((NUM_FEATURES, VPAD, D), lambda b, i: (0, 0, 0)),
        ],
        out_specs=pl.BlockSpec((1, H, 8, S, D), lambda b, i: (b, 0, i, 0, 0)),
        out_shape=out_sds,
    )(idx_raw, vtabs)
    return k_out, v_out


def kernel(inputs, emb0, emb1):
    scale = float(D) ** 0.5
    idx_perm = jnp.transpose(
        inputs.reshape(NUM_FEATURES, B, NW, ROWS_PER_W, S),
        (2, 0, 1, 3, 4)).reshape(NW, NUM_FEATURES * B * ROWS_PER_W, S)
    kt0 = emb0[:, :D] * scale
    kt1 = emb1[:, :D] * scale
    vtabs = jnp.zeros((NUM_FEATURES, VPAD, D), jnp.float32)
    vtabs = vtabs.at[0, :emb0.shape[0]].set(emb0[:, D:] * scale)
    vtabs = vtabs.at[1, :emb1.shape[0]].set(emb1[:, D:] * scale)
    k_out, v_out = _tree_rel_pos(idx_perm, inputs, kt0, kt1, vtabs)
    return (k_out, v_out)


# speedup vs baseline: 1.2061x; 1.0313x over previous
import functools

import jax
import jax.numpy as jnp
from jax import lax
from jax.experimental import pallas as pl
from jax.experimental.pallas import tpu as pltpu
from jax.experimental.pallas import tpu_sc as plsc

NUM_FEATURES = 2
B = 2
S = 128
D = 64
REPS = 4
H = NUM_FEATURES * REPS
NW = 32
ROWS_PER_W = S // NW
BLK = 16


def _sc_gather_body(idx_hbm, kv0, kv1, inter, idx_v, b0, b1, b2,
                    gs0, gs1, gs2, ss0, ss1, ss2):
    wid = lax.axis_index("s") * 2 + lax.axis_index("c")
    bufs = (b0, b1, b2)
    gsems = (gs0, gs1, gs2)
    ssems = (ss0, ss1, ss2)
    tables = (kv0, kv1)
    units = [(f, b, half) for f in range(NUM_FEATURES) for b in range(B)
             for half in range(2)]
    pltpu.sync_copy(idx_hbm.at[wid], idx_v)
    gathers = [None] * 3
    scatters = [None] * 3

    def issue_gathers(t):
        f, b, half = units[t]
        slot = t % 3
        gathers[slot] = [
            pltpu.async_copy(
                tables[f].at[idx_v.at[(f * B + b) * ROWS_PER_W + 2 * half + c]],
                bufs[slot].at[c], gsems[slot])
            for c in range(2)
        ]

    for t in range(2):
        issue_gathers(t)
    for t in range(len(units)):
        f, b, half = units[t]
        slot = t % 3
        for g in gathers[slot]:
            g.wait()
        i0 = wid * ROWS_PER_W + 2 * half
        scatters[slot] = pltpu.async_copy(
            bufs[slot], inter.at[f, b, pl.ds(i0, 2), :, :],
            ssems[slot])
        nxt = t + 2
        if nxt < len(units):
            nslot = nxt % 3
            if scatters[nslot] is not None:
                scatters[nslot].wait()
                scatters[nslot] = None
            issue_gathers(nxt)
    for s in scatters:
        if s is not None:
            s.wait()


def _tc_replicate_body(inter_ref, k_ref, v_ref):
    for f in range(NUM_FEATURES):
        x = inter_ref[f, 0]
        k = x[:, :, :D]
        v = x[:, :, D:]
        for r in range(REPS):
            h = REPS * f + r
            k_ref[0, h] = k
            v_ref[0, h] = v


@jax.jit
def _tree_rel_pos(idx_perm, kv0, kv1):
    mesh = plsc.VectorSubcoreMesh(core_axis_name="c", subcore_axis_name="s")
    inter_sds = jax.ShapeDtypeStruct((NUM_FEATURES, B, S, S, 2 * D),
                                     jnp.float32)
    buf = pltpu.VMEM((2, S, 2 * D), jnp.float32)
    sc_run = functools.partial(
        pl.kernel,
        out_type=inter_sds,
        mesh=mesh,
        scratch_types=[pltpu.VMEM((NUM_FEATURES * B * ROWS_PER_W, S),
                                  jnp.int32)]
        + [buf] * 3 + [pltpu.SemaphoreType.DMA] * 6,
    )(_sc_gather_body)
    inter = sc_run(idx_perm, kv0, kv1)

    out_sds = jax.ShapeDtypeStruct((B, H, S, S, D), jnp.float32)
    k_out, v_out = pl.pallas_call(
        _tc_replicate_body,
        grid=(B, S // BLK),
        in_specs=[pl.BlockSpec((NUM_FEATURES, 1, BLK, S, 2 * D),
                               lambda b, i: (0, b, i, 0, 0))],
        out_specs=[
            pl.BlockSpec((1, H, BLK, S, D), lambda b, i: (b, 0, i, 0, 0)),
            pl.BlockSpec((1, H, BLK, S, D), lambda b, i: (b, 0, i, 0, 0)),
        ],
        out_shape=[out_sds, out_sds],
    )(inter)
    return k_out, v_out


def kernel(inputs, emb0, emb1):
    scale = float(D) ** 0.5
    idx_perm = jnp.transpose(
        inputs.reshape(NUM_FEATURES, B, NW, ROWS_PER_W, S),
        (2, 0, 1, 3, 4)).reshape(NW, NUM_FEATURES * B * ROWS_PER_W, S)
    k_out, v_out = _tree_rel_pos(idx_perm, emb0 * scale, emb1 * scale)
    return (k_out, v_out)
